# baseline (speedup 1.0000x reference)
import math

import jax
import jax.numpy as jnp
import numpy as np
from jax.experimental import pallas as pl
from jax.experimental.pallas import tpu as pltpu

WIDTH_R = 0.5
L = 16
K = 32
TOKEN_C = 64
HEAD = 2
NUM_CLASSES = 50
CS = [int(WIDTH_R * c) for c in [32, 64, 128, 256, 256]]
B, CIN, N = 4, 22, 16384
NS = L * K
HD = TOKEN_C // HEAD
SCALE = 1.0 / float(np.sqrt(HD))
VPAD = 128
BL = B * L
NSB = B * NS

FOLD = 8
NF = N // FOLD

NEG = -1e30

RIM_SPECS = [
    ("stage1", "vt1", CS[0], CS[0], False, False),
    ("stage1", "vt2", CS[0], CS[1], True, True),
    ("stage2", "vt1", CS[1], CS[1], False, True),
    ("stage2", "vt2", CS[1], CS[2], True, True),
    ("stage3", "vt1", CS[2], CS[2], False, True),
    ("stage3", "vt2", CS[2], CS[3], True, True),
    ("stage4", "vt1", CS[3], CS[3], False, True),
    ("stage4", "vt2", CS[3], CS[4], False, True),
]
RIM_KEYS = ["dyn1", "dyn2", "tq", "tk", "tv", "tp", "tf1", "tf2",
            "pq", "pk", "pv"]

TILE_N = 8192


def _mm(a, b):
    return jax.lax.dot_general(a, b, (((1,), (0,)), ((), ())),
                               preferred_element_type=jnp.float32)


def _mm_tt(a, b):
    return jax.lax.dot_general(a, b, (((0,), (0,)), ((), ())),
                               preferred_element_type=jnp.float32)


def _mm_nt(a, b):
    return jax.lax.dot_general(a, b, (((1,), (1,)), ((), ())),
                               preferred_element_type=jnp.float32)


def _block_kv(kr, vr, outc):
    krs = kr * SCALE
    zk = jnp.zeros((HD, L), jnp.float32)
    kb = jnp.concatenate(
        [jnp.concatenate([krs[0:HD], zk], axis=1),
         jnp.concatenate([zk, krs[HD:TOKEN_C]], axis=1)], axis=0)
    half = outc // HEAD
    zt = jnp.zeros((half, L), jnp.float32)
    zb = jnp.zeros((outc - half, L), jnp.float32)
    vb = jnp.concatenate(
        [jnp.concatenate([vr[0:half], zt], axis=1),
         jnp.concatenate([zb, vr[half:outc]], axis=1)], axis=0)
    return kb, vb


def _point_attn(kb, vb, fb, q):
    logits = _mm_tt(kb, q)
    parts = []
    for h in range(HEAD):
        lg = logits[h * L:(h + 1) * L]
        lg = lg - jnp.max(lg, axis=0, keepdims=True)
        e = jnp.exp(lg)
        parts.append(e / jnp.sum(e, axis=0, keepdims=True))
    a = jnp.concatenate(parts, axis=0)
    return fb + _mm(vb, a)


def _phase_a_body(x_ref, cf_ref, stem1, stem2, rws, k_out, v_out):
    xs = [x_ref[b] for b in range(B)]
    x88 = jnp.concatenate(xs, axis=0)
    iota_n = jax.lax.broadcasted_iota(jnp.int32, (1, N), 1)
    iota_f = (jax.lax.broadcasted_iota(jnp.int32, (FOLD, NF), 0) * NF
              + jax.lax.broadcasted_iota(jnp.int32, (FOLD, NF), 1))

    centers = []
    for b in range(B):
        cxf = cf_ref[b, 0]
        cyf = cf_ref[b, 1]
        czf = cf_ref[b, 2]
        dists = jnp.full((FOLD, NF), 1e10, jnp.float32)
        prev = jnp.int32(0)
        cxs, cys, czs = [], [], []
        for i in range(1, L + 1):
            sel_mask = iota_f == prev
            lx = jnp.sum(jnp.where(sel_mask, cxf, 0.0))
            ly = jnp.sum(jnp.where(sel_mask, cyf, 0.0))
            lz = jnp.sum(jnp.where(sel_mask, czf, 0.0))
            cxs.append(lx)
            cys.append(ly)
            czs.append(lz)
            if i < L:
                d = (cxf - lx) ** 2 + (cyf - ly) ** 2 + (czf - lz) ** 2
                dists = jnp.minimum(dists, d)
                m = jnp.max(dists)
                prev = jnp.min(
                    jnp.where(dists == m, iota_f, N)).astype(jnp.int32)
        centers.append((cxs, cys, czs))

    dblocks = []
    for b in range(B):
        cxs, cys, czs = centers[b]
        cxc = jnp.concatenate([v.reshape(1, 1) for v in cxs], axis=0)
        cyc = jnp.concatenate([v.reshape(1, 1) for v in cys], axis=0)
        czc = jnp.concatenate([v.reshape(1, 1) for v in czs], axis=0)
        xb = xs[b]
        dblocks.append((xb[0:1, :] - cxc) ** 2 + (xb[1:2, :] - cyc) ** 2
                       + (xb[2:3, :] - czc) ** 2)
    D = jnp.concatenate(dblocks, axis=0)

    glist = []
    for _k in range(K):
        rowmin = jnp.min(D, axis=1, keepdims=True)
        cand = jnp.where(D == rowmin, iota_n, N)
        sel = jnp.min(cand, axis=1, keepdims=True).astype(jnp.int32)
        onehot = iota_n == sel
        g = _mm_nt(x88, jnp.where(onehot, 1.0, 0.0))
        g = jnp.concatenate(
            [g[b * CIN:(b + 1) * CIN, b * L:(b + 1) * L]
             for b in range(B)], axis=1)
        glist.append(g)
        D = jnp.where(onehot, jnp.float32(np.inf), D)
    xg = jnp.concatenate(glist, axis=1)

    fs = _mm(stem2, jnp.maximum(_mm(stem1, xg), 0.0))

    r_tok = jax.lax.broadcasted_iota(jnp.int32, (BL, BL), 0) // L
    c_tok = jax.lax.broadcasted_iota(jnp.int32, (BL, BL), 1) // L
    m_tok = jnp.where(r_tok == c_tok, 0.0, NEG)
    r_fs = jax.lax.broadcasted_iota(jnp.int32, (HEAD * BL, NSB), 0) // (HEAD * L)
    c_fs = (jax.lax.broadcasted_iota(jnp.int32, (HEAD * BL, NSB), 1) // L) % B
    m_fs01 = jnp.where(r_fs == c_fs, 1.0, 0.0)

    tokens = None
    for r, (_, _, inc, outc, has_fb, _) in enumerate(RIM_SPECS):
        w = rws[r]
        t0 = fs[:, 0:BL]
        for kk in range(1, K):
            t0 = jnp.maximum(t0, fs[:, kk * BL:(kk + 1) * BL])
        t = _mm(w["dyn2"], jnp.maximum(_mm(w["dyn1"], t0), 0.0))
        if tokens is not None:
            t = t + tokens

        q = _mm(w["tq"], t)
        k_ = _mm(w["tk"], t)
        v = _mm(w["tv"], t)
        outs = []
        for h in range(HEAD):
            qh = q[h * HD:(h + 1) * HD]
            kh = k_[h * HD:(h + 1) * HD]
            vh = v[h * HD:(h + 1) * HD]
            logits = _mm_tt(qh, kh) * SCALE + m_tok
            logits = logits - jnp.max(logits, axis=1, keepdims=True)
            e = jnp.exp(logits)
            a = e / jnp.sum(e, axis=1, keepdims=True)
            outs.append(_mm_nt(vh, a))
        o = jnp.concatenate(outs, axis=0)
        t = t + _mm(w["tp"], o)
        t = t + _mm(w["tf2"], jnp.maximum(_mm(w["tf1"], t), 0.0))
        tokens = t

        kr = _mm(w["pk"], t)
        vr = _mm(w["pv"], t)
        kbs, vbs = [], []
        for b in range(B):
            kb, vb = _block_kv(kr[:, b * L:(b + 1) * L],
                               vr[:, b * L:(b + 1) * L], outc)
            kbs.append(kb)
            vbs.append(vb)
            k_out[b, r] = kb
            if outc < VPAD:
                vb_store = jnp.concatenate(
                    [vb, jnp.zeros((VPAD - outc, HEAD * L), jnp.float32)],
                    axis=0)
            else:
                vb_store = vb
            v_out[b, r] = vb_store
        kbig = jnp.concatenate(kbs, axis=1)
        vbig = jnp.concatenate(vbs, axis=1)

        fb = _mm(w["fb"], fs) if has_fb else fs
        qp = _mm(w["pq"], fb)
        logits = _mm_tt(kbig, qp)
        parts = []
        for g in range(HEAD * B):
            lg = logits[g * L:(g + 1) * L]
            lg = lg - jnp.max(lg, axis=0, keepdims=True)
            e = jnp.exp(lg)
            parts.append(e / jnp.sum(e, axis=0, keepdims=True))
        a = jnp.concatenate(parts, axis=0) * m_fs01
        fs = fb + _mm(vbig, a)


def _flat_weights_a(params):
    ws = [params["stem1"], params["stem2"]]
    for (stage, vt, inc, outc, has_fb, _) in RIM_SPECS:
        p = params[stage][vt]
        ws.extend(p[kk] for kk in RIM_KEYS)
        if has_fb:
            ws.append(p["fb"])
    return ws


def _flat_weights_b(params):
    ws = [params["stem1"], params["stem2"]]
    ws.extend(params[s][vt]["pq"] for (s, vt, *_r) in RIM_SPECS)
    ws.extend(params[s][vt]["fb"]
              for (s, vt, _i, _o, has_fb, _t) in RIM_SPECS if has_fb)
    ws.extend([params["cls1"], params["cls2"]])
    return ws


def _full_spec(arr):
    nd = arr.ndim
    return pl.BlockSpec(arr.shape, lambda *_: (0,) * nd)


def _fused_kernel(xt_ref, xf_ref, cf_ref, *refs):
    w_refs = refs[:-3]
    out_ref, k_scr, v_scr = refs[-3], refs[-2], refs[-1]

    it = iter(w_refs)
    stem1 = next(it)[...]
    stem2 = next(it)[...]
    rws = []
    for (_, _, inc, outc, has_fb, _) in RIM_SPECS:
        w = {kk: next(it)[...] for kk in RIM_KEYS}
        if has_fb:
            w["fb"] = next(it)[...]
        rws.append(w)
    cls1 = next(it)[...]
    cls2 = next(it)[...]

    b = pl.program_id(0)
    t = pl.program_id(1)

    @pl.when(jnp.logical_and(b == 0, t == 0))
    def _():
        _phase_a_body(xf_ref, cf_ref, stem1, stem2, rws, k_scr, v_scr)

    xb = xt_ref[0]
    f = _mm(stem2, jnp.maximum(_mm(stem1, xb), 0.0))
    for r, (_, _, inc, outc, has_fb, _) in enumerate(RIM_SPECS):
        kb = k_scr[b, r]
        vb = v_scr[b, r, 0:outc, :]
        if has_fb:
            q = _mm(_mm(rws[r]["pq"], rws[r]["fb"]), f)
            fb = _mm(rws[r]["fb"], f)
        else:
            fb = f
            q = _mm(rws[r]["pq"], fb)
        f = _point_attn(kb, vb, fb, q)
    out_ref[0] = _mm(cls2, jnp.maximum(_mm(cls1, f), 0.0))


def kernel(x, params):
    coords_folded = x[:, :3, :].reshape(B, 3, FOLD, NF)
    ws = _flat_weights_a(params) + [params["cls1"], params["cls2"]]
    nr = len(RIM_SPECS)
    nt = N // TILE_N
    in_specs = [
        pl.BlockSpec((1, CIN, TILE_N), lambda b, t: (b, 0, t)),
        pl.BlockSpec((B, CIN, N), lambda b, t: (0, 0, 0)),
        pl.BlockSpec((B, 3, FOLD, NF), lambda b, t: (0, 0, 0, 0)),
    ]
    in_specs += [_full_spec(w) for w in ws]
    out = pl.pallas_call(
        _fused_kernel,
        grid=(B, nt),
        in_specs=in_specs,
        out_specs=pl.BlockSpec((1, NUM_CLASSES, TILE_N),
                               lambda b, t: (b, 0, t)),
        out_shape=jax.ShapeDtypeStruct((B, NUM_CLASSES, N), jnp.float32),
        scratch_shapes=[
            pltpu.VMEM((B, nr, TOKEN_C, HEAD * L), jnp.float32),
            pltpu.VMEM((B, nr, VPAD, HEAD * L), jnp.float32),
        ],
        compiler_params=pltpu.CompilerParams(
            dimension_semantics=("arbitrary", "arbitrary")),
    )(x, x, coords_folded, *ws)
    return out


# speedup vs baseline: 1.2360x; 1.2360x over previous
import math

import jax
import jax.numpy as jnp
import numpy as np
from jax.experimental import pallas as pl
from jax.experimental.pallas import tpu as pltpu

WIDTH_R = 0.5
L = 16
K = 32
TOKEN_C = 64
HEAD = 2
NUM_CLASSES = 50
CS = [int(WIDTH_R * c) for c in [32, 64, 128, 256, 256]]
B, CIN, N = 4, 22, 16384
NS = L * K
HD = TOKEN_C // HEAD
SCALE = 1.0 / float(np.sqrt(HD))
VPAD = 128
BL = B * L
NSB = B * NS

FOLD = 8
NF = N // FOLD

NEG = -1e30

RIM_SPECS = [
    ("stage1", "vt1", CS[0], CS[0], False, False),
    ("stage1", "vt2", CS[0], CS[1], True, True),
    ("stage2", "vt1", CS[1], CS[1], False, True),
    ("stage2", "vt2", CS[1], CS[2], True, True),
    ("stage3", "vt1", CS[2], CS[2], False, True),
    ("stage3", "vt2", CS[2], CS[3], True, True),
    ("stage4", "vt1", CS[3], CS[3], False, True),
    ("stage4", "vt2", CS[3], CS[4], False, True),
]
RIM_KEYS = ["dyn1", "dyn2", "tq", "tk", "tv", "tp", "tf1", "tf2",
            "pq", "pk", "pv"]

TILE_N = 8192


def _mm(a, b):
    return jax.lax.dot_general(a, b, (((1,), (0,)), ((), ())),
                               preferred_element_type=jnp.float32)


def _mm_tt(a, b):
    return jax.lax.dot_general(a, b, (((0,), (0,)), ((), ())),
                               preferred_element_type=jnp.float32)


def _mm_nt(a, b):
    return jax.lax.dot_general(a, b, (((1,), (1,)), ((), ())),
                               preferred_element_type=jnp.float32)


def _block_kv(kr, vr, outc):
    krs = kr * SCALE
    zk = jnp.zeros((HD, L), jnp.float32)
    kb = jnp.concatenate(
        [jnp.concatenate([krs[0:HD], zk], axis=1),
         jnp.concatenate([zk, krs[HD:TOKEN_C]], axis=1)], axis=0)
    half = outc // HEAD
    zt = jnp.zeros((half, L), jnp.float32)
    zb = jnp.zeros((outc - half, L), jnp.float32)
    vb = jnp.concatenate(
        [jnp.concatenate([vr[0:half], zt], axis=1),
         jnp.concatenate([zb, vr[half:outc]], axis=1)], axis=0)
    return kb, vb


def _point_attn(kb, vb, fb, q):
    logits = _mm_tt(kb, q)
    parts = []
    for h in range(HEAD):
        lg = logits[h * L:(h + 1) * L]
        lg = lg - jnp.max(lg, axis=0, keepdims=True)
        e = jnp.exp(lg)
        parts.append(e / jnp.sum(e, axis=0, keepdims=True))
    a = jnp.concatenate(parts, axis=0)
    return fb + _mm(vb, a)


def _tokens_kernel(x_ref, cf_ref, *refs):
    w_refs = refs[:-2]
    k_out, v_out = refs[-2], refs[-1]

    it = iter(w_refs)
    stem1 = next(it)[...]
    stem2 = next(it)[...]
    rws = []
    for (_, _, inc, outc, has_fb, _) in RIM_SPECS:
        w = {kk: next(it)[...] for kk in RIM_KEYS}
        if has_fb:
            w["fb"] = next(it)[...]
        rws.append(w)

    xs = [x_ref[b] for b in range(B)]
    x88 = jnp.concatenate(xs, axis=0)
    iota_n = jax.lax.broadcasted_iota(jnp.int32, (1, N), 1)
    iota_f = (jax.lax.broadcasted_iota(jnp.int32, (FOLD, NF), 0) * NF
              + jax.lax.broadcasted_iota(jnp.int32, (FOLD, NF), 1))

    centers = []
    for b in range(B):
        cxf = cf_ref[b, 0]
        cyf = cf_ref[b, 1]
        czf = cf_ref[b, 2]
        dists = jnp.full((FOLD, NF), 1e10, jnp.float32)
        prev = jnp.int32(0)
        cxs, cys, czs = [], [], []
        for i in range(1, L + 1):
            sel_mask = iota_f == prev
            lx = jnp.sum(jnp.where(sel_mask, cxf, 0.0))
            ly = jnp.sum(jnp.where(sel_mask, cyf, 0.0))
            lz = jnp.sum(jnp.where(sel_mask, czf, 0.0))
            cxs.append(lx)
            cys.append(ly)
            czs.append(lz)
            if i < L:
                d = (cxf - lx) ** 2 + (cyf - ly) ** 2 + (czf - lz) ** 2
                dists = jnp.minimum(dists, d)
                m = jnp.max(dists)
                prev = jnp.min(
                    jnp.where(dists == m, iota_f, N)).astype(jnp.int32)
        centers.append((cxs, cys, czs))

    dblocks = []
    for b in range(B):
        cxs, cys, czs = centers[b]
        cxc = jnp.concatenate([v.reshape(1, 1) for v in cxs], axis=0)
        cyc = jnp.concatenate([v.reshape(1, 1) for v in cys], axis=0)
        czc = jnp.concatenate([v.reshape(1, 1) for v in czs], axis=0)
        xb = xs[b]
        dblocks.append((xb[0:1, :] - cxc) ** 2 + (xb[1:2, :] - cyc) ** 2
                       + (xb[2:3, :] - czc) ** 2)
    D = jnp.concatenate(dblocks, axis=0)

    glist = []
    for _k in range(K):
        rowmin = jnp.min(D, axis=1, keepdims=True)
        cand = jnp.where(D == rowmin, iota_n, N)
        sel = jnp.min(cand, axis=1, keepdims=True).astype(jnp.int32)
        onehot = iota_n == sel
        g = _mm_nt(x88, jnp.where(onehot, 1.0, 0.0))
        g = jnp.concatenate(
            [g[b * CIN:(b + 1) * CIN, b * L:(b + 1) * L]
             for b in range(B)], axis=1)
        glist.append(g)
        D = jnp.where(onehot, jnp.float32(np.inf), D)
    xg = jnp.concatenate(glist, axis=1)

    fs = _mm(stem2, jnp.maximum(_mm(stem1, xg), 0.0))

    r_tok = jax.lax.broadcasted_iota(jnp.int32, (BL, BL), 0) // L
    c_tok = jax.lax.broadcasted_iota(jnp.int32, (BL, BL), 1) // L
    m_tok = jnp.where(r_tok == c_tok, 0.0, NEG)
    r_fs = jax.lax.broadcasted_iota(jnp.int32, (HEAD * BL, NSB), 0) // (HEAD * L)
    c_fs = (jax.lax.broadcasted_iota(jnp.int32, (HEAD * BL, NSB), 1) // L) % B
    m_fs01 = jnp.where(r_fs == c_fs, 1.0, 0.0)

    tokens = None
    for r, (_, _, inc, outc, has_fb, _) in enumerate(RIM_SPECS):
        w = rws[r]
        t0 = fs[:, 0:BL]
        for kk in range(1, K):
            t0 = jnp.maximum(t0, fs[:, kk * BL:(kk + 1) * BL])
        t = _mm(w["dyn2"], jnp.maximum(_mm(w["dyn1"], t0), 0.0))
        if tokens is not None:
            t = t + tokens

        q = _mm(w["tq"], t)
        k_ = _mm(w["tk"], t)
        v = _mm(w["tv"], t)
        outs = []
        for h in range(HEAD):
            qh = q[h * HD:(h + 1) * HD]
            kh = k_[h * HD:(h + 1) * HD]
            vh = v[h * HD:(h + 1) * HD]
            logits = _mm_tt(qh, kh) * SCALE + m_tok
            logits = logits - jnp.max(logits, axis=1, keepdims=True)
            e = jnp.exp(logits)
            a = e / jnp.sum(e, axis=1, keepdims=True)
            outs.append(_mm_nt(vh, a))
        o = jnp.concatenate(outs, axis=0)
        t = t + _mm(w["tp"], o)
        t = t + _mm(w["tf2"], jnp.maximum(_mm(w["tf1"], t), 0.0))
        tokens = t

        kr = _mm(w["pk"], t)
        vr = _mm(w["pv"], t)
        kbs, vbs = [], []
        for b in range(B):
            kb, vb = _block_kv(kr[:, b * L:(b + 1) * L],
                               vr[:, b * L:(b + 1) * L], outc)
            kbs.append(kb)
            vbs.append(vb)
            k_out[b, r] = kb
            if outc < VPAD:
                vb_store = jnp.concatenate(
                    [vb, jnp.zeros((VPAD - outc, HEAD * L), jnp.float32)],
                    axis=0)
            else:
                vb_store = vb
            v_out[b, r] = vb_store
        kbig = jnp.concatenate(kbs, axis=1)
        vbig = jnp.concatenate(vbs, axis=1)

        fb = _mm(w["fb"], fs) if has_fb else fs
        qp = _mm(w["pq"], fb)
        logits = _mm_tt(kbig, qp)
        parts = []
        for g in range(HEAD * B):
            lg = logits[g * L:(g + 1) * L]
            lg = lg - jnp.max(lg, axis=0, keepdims=True)
            e = jnp.exp(lg)
            parts.append(e / jnp.sum(e, axis=0, keepdims=True))
        a = jnp.concatenate(parts, axis=0) * m_fs01
        fs = fb + _mm(vbig, a)


def _dense_kernel(x_ref, kall_ref, vall_ref, *refs):
    it = iter(refs[:-1])
    out_ref = refs[-1]
    stem1 = next(it)[...]
    stem2 = next(it)[...]
    pqs = [next(it)[...] for _ in RIM_SPECS]
    fbs = {r: next(it)[...] for r, s in enumerate(RIM_SPECS) if s[4]}
    cls1 = next(it)[...]
    cls2 = next(it)[...]

    xb = x_ref[0]
    f = _mm(stem2, jnp.maximum(_mm(stem1, xb), 0.0))
    for r, (_, _, inc, outc, has_fb, _) in enumerate(RIM_SPECS):
        kb = kall_ref[0, r]
        vb = vall_ref[0, r, 0:outc, :]
        if has_fb:
            q = _mm(_mm(pqs[r], fbs[r]), f)
            fb = _mm(fbs[r], f)
        else:
            fb = f
            q = _mm(pqs[r], fb)
        f = _point_attn(kb, vb, fb, q)
    out_ref[0] = _mm(cls2, jnp.maximum(_mm(cls1, f), 0.0))


def _flat_weights_a(params):
    ws = [params["stem1"], params["stem2"]]
    for (stage, vt, inc, outc, has_fb, _) in RIM_SPECS:
        p = params[stage][vt]
        ws.extend(p[kk] for kk in RIM_KEYS)
        if has_fb:
            ws.append(p["fb"])
    return ws


def _flat_weights_b(params):
    ws = [params["stem1"], params["stem2"]]
    ws.extend(params[s][vt]["pq"] for (s, vt, *_r) in RIM_SPECS)
    ws.extend(params[s][vt]["fb"]
              for (s, vt, _i, _o, has_fb, _t) in RIM_SPECS if has_fb)
    ws.extend([params["cls1"], params["cls2"]])
    return ws


def _full_spec(arr):
    nd = arr.ndim
    return pl.BlockSpec(arr.shape, lambda *_: (0,) * nd)


def kernel(x, params):
    coords_folded = x[:, :3, :].reshape(B, 3, FOLD, NF)
    wa = _flat_weights_a(params)
    in_specs_a = [
        pl.BlockSpec((B, CIN, N), lambda: (0, 0, 0)),
        pl.BlockSpec((B, 3, FOLD, NF), lambda: (0, 0, 0, 0)),
    ]
    in_specs_a += [_full_spec(w) for w in wa]
    nr = len(RIM_SPECS)
    k_all, v_all = pl.pallas_call(
        _tokens_kernel,
        grid=(),
        in_specs=in_specs_a,
        out_specs=[
            pl.BlockSpec((B, nr, TOKEN_C, HEAD * L), lambda: (0, 0, 0, 0)),
            pl.BlockSpec((B, nr, VPAD, HEAD * L), lambda: (0, 0, 0, 0)),
        ],
        out_shape=[
            jax.ShapeDtypeStruct((B, nr, TOKEN_C, HEAD * L), jnp.float32),
            jax.ShapeDtypeStruct((B, nr, VPAD, HEAD * L), jnp.float32),
        ],
    )(x, coords_folded, *wa)

    wb = _flat_weights_b(params)
    nt = N // TILE_N
    in_specs_b = [
        pl.BlockSpec((1, CIN, TILE_N), lambda b, t: (b, 0, t)),
        pl.BlockSpec((1, nr, TOKEN_C, HEAD * L), lambda b, t: (b, 0, 0, 0)),
        pl.BlockSpec((1, nr, VPAD, HEAD * L), lambda b, t: (b, 0, 0, 0)),
    ]
    in_specs_b += [_full_spec(w) for w in wb]
    out = pl.pallas_call(
        _dense_kernel,
        grid=(B, nt),
        in_specs=in_specs_b,
        out_specs=pl.BlockSpec((1, NUM_CLASSES, TILE_N), lambda b, t: (b, 0, t)),
        out_shape=jax.ShapeDtypeStruct((B, NUM_CLASSES, N), jnp.float32),
        compiler_params=pltpu.CompilerParams(
            dimension_semantics=("arbitrary", "arbitrary")),
    )(x, k_all, v_all, *wb)
    return out


# speedup vs baseline: 1.2656x; 1.0239x over previous
import math

import jax
import jax.numpy as jnp
import numpy as np
from jax.experimental import pallas as pl
from jax.experimental.pallas import tpu as pltpu

WIDTH_R = 0.5
L = 16
K = 32
TOKEN_C = 64
HEAD = 2
NUM_CLASSES = 50
CS = [int(WIDTH_R * c) for c in [32, 64, 128, 256, 256]]
B, CIN, N = 4, 22, 16384
NS = L * K
HD = TOKEN_C // HEAD
SCALE = 1.0 / float(np.sqrt(HD))
VPAD = 128
BL = B * L
NSB = B * NS

FOLD = 8
NF = N // FOLD

NEG = -1e30

RIM_SPECS = [
    ("stage1", "vt1", CS[0], CS[0], False, False),
    ("stage1", "vt2", CS[0], CS[1], True, True),
    ("stage2", "vt1", CS[1], CS[1], False, True),
    ("stage2", "vt2", CS[1], CS[2], True, True),
    ("stage3", "vt1", CS[2], CS[2], False, True),
    ("stage3", "vt2", CS[2], CS[3], True, True),
    ("stage4", "vt1", CS[3], CS[3], False, True),
    ("stage4", "vt2", CS[3], CS[4], False, True),
]
RIM_KEYS = ["dyn1", "dyn2", "tq", "tk", "tv", "tp", "tf1", "tf2",
            "pq", "pk", "pv"]

TILE_N = 16384


def _mm(a, b):
    return jax.lax.dot_general(a, b, (((1,), (0,)), ((), ())),
                               preferred_element_type=jnp.float32)


def _mm_tt(a, b):
    return jax.lax.dot_general(a, b, (((0,), (0,)), ((), ())),
                               preferred_element_type=jnp.float32)


def _mm_nt(a, b):
    return jax.lax.dot_general(a, b, (((1,), (1,)), ((), ())),
                               preferred_element_type=jnp.float32)


def _block_kv(kr, vr, outc):
    krs = kr * SCALE
    zk = jnp.zeros((HD, L), jnp.float32)
    kb = jnp.concatenate(
        [jnp.concatenate([krs[0:HD], zk], axis=1),
         jnp.concatenate([zk, krs[HD:TOKEN_C]], axis=1)], axis=0)
    half = outc // HEAD
    zt = jnp.zeros((half, L), jnp.float32)
    zb = jnp.zeros((outc - half, L), jnp.float32)
    vb = jnp.concatenate(
        [jnp.concatenate([vr[0:half], zt], axis=1),
         jnp.concatenate([zb, vr[half:outc]], axis=1)], axis=0)
    return kb, vb


def _point_attn(kb, vb, fb, q):
    logits = _mm_tt(kb, q)
    parts = []
    for h in range(HEAD):
        lg = logits[h * L:(h + 1) * L]
        lg = lg - jnp.max(lg, axis=0, keepdims=True)
        e = jnp.exp(lg)
        parts.append(e / jnp.sum(e, axis=0, keepdims=True))
    a = jnp.concatenate(parts, axis=0)
    return fb + _mm(vb, a)


def _tokens_kernel(x_ref, cf_ref, *refs):
    w_refs = refs[:-2]
    k_out, v_out = refs[-2], refs[-1]

    it = iter(w_refs)
    stem1 = next(it)[...]
    stem2 = next(it)[...]
    rws = []
    for (_, _, inc, outc, has_fb, _) in RIM_SPECS:
        w = {kk: next(it)[...] for kk in RIM_KEYS}
        if has_fb:
            w["fb"] = next(it)[...]
        rws.append(w)

    xs = [x_ref[b] for b in range(B)]
    x88 = jnp.concatenate(xs, axis=0)
    iota_n = jax.lax.broadcasted_iota(jnp.int32, (1, N), 1)
    iota_f = (jax.lax.broadcasted_iota(jnp.int32, (FOLD, NF), 0) * NF
              + jax.lax.broadcasted_iota(jnp.int32, (FOLD, NF), 1))

    centers = []
    for b in range(B):
        cxf = cf_ref[b, 0]
        cyf = cf_ref[b, 1]
        czf = cf_ref[b, 2]
        dists = jnp.full((FOLD, NF), 1e10, jnp.float32)
        prev = jnp.int32(0)
        cxs, cys, czs = [], [], []
        for i in range(1, L + 1):
            sel_mask = iota_f == prev
            lx = jnp.sum(jnp.where(sel_mask, cxf, 0.0))
            ly = jnp.sum(jnp.where(sel_mask, cyf, 0.0))
            lz = jnp.sum(jnp.where(sel_mask, czf, 0.0))
            cxs.append(lx)
            cys.append(ly)
            czs.append(lz)
            if i < L:
                d = (cxf - lx) ** 2 + (cyf - ly) ** 2 + (czf - lz) ** 2
                dists = jnp.minimum(dists, d)
                m = jnp.max(dists)
                prev = jnp.min(
                    jnp.where(dists == m, iota_f, N)).astype(jnp.int32)
        centers.append((cxs, cys, czs))

    dblocks = []
    for b in range(B):
        cxs, cys, czs = centers[b]
        cxc = jnp.concatenate([v.reshape(1, 1) for v in cxs], axis=0)
        cyc = jnp.concatenate([v.reshape(1, 1) for v in cys], axis=0)
        czc = jnp.concatenate([v.reshape(1, 1) for v in czs], axis=0)
        xb = xs[b]
        dblocks.append((xb[0:1, :] - cxc) ** 2 + (xb[1:2, :] - cyc) ** 2
                       + (xb[2:3, :] - czc) ** 2)
    D = jnp.concatenate(dblocks, axis=0)

    glist = []
    for _k in range(K):
        rowmin = jnp.min(D, axis=1, keepdims=True)
        cand = jnp.where(D == rowmin, iota_n, N)
        sel = jnp.min(cand, axis=1, keepdims=True).astype(jnp.int32)
        onehot = iota_n == sel
        g = _mm_nt(x88, jnp.where(onehot, 1.0, 0.0))
        g = jnp.concatenate(
            [g[b * CIN:(b + 1) * CIN, b * L:(b + 1) * L]
             for b in range(B)], axis=1)
        glist.append(g)
        D = jnp.where(onehot, jnp.float32(np.inf), D)
    xg = jnp.concatenate(glist, axis=1)

    fs = _mm(stem2, jnp.maximum(_mm(stem1, xg), 0.0))

    r_tok = jax.lax.broadcasted_iota(jnp.int32, (BL, BL), 0) // L
    c_tok = jax.lax.broadcasted_iota(jnp.int32, (BL, BL), 1) // L
    m_tok = jnp.where(r_tok == c_tok, 0.0, NEG)
    r_fs = jax.lax.broadcasted_iota(jnp.int32, (HEAD * BL, NSB), 0) // (HEAD * L)
    c_fs = (jax.lax.broadcasted_iota(jnp.int32, (HEAD * BL, NSB), 1) // L) % B
    m_fs01 = jnp.where(r_fs == c_fs, 1.0, 0.0)

    tokens = None
    for r, (_, _, inc, outc, has_fb, _) in enumerate(RIM_SPECS):
        w = rws[r]
        t0 = fs[:, 0:BL]
        for kk in range(1, K):
            t0 = jnp.maximum(t0, fs[:, kk * BL:(kk + 1) * BL])
        t = _mm(w["dyn2"], jnp.maximum(_mm(w["dyn1"], t0), 0.0))
        if tokens is not None:
            t = t + tokens

        q = _mm(w["tq"], t)
        k_ = _mm(w["tk"], t)
        v = _mm(w["tv"], t)
        outs = []
        for h in range(HEAD):
            qh = q[h * HD:(h + 1) * HD]
            kh = k_[h * HD:(h + 1) * HD]
            vh = v[h * HD:(h + 1) * HD]
            logits = _mm_tt(qh, kh) * SCALE + m_tok
            logits = logits - jnp.max(logits, axis=1, keepdims=True)
            e = jnp.exp(logits)
            a = e / jnp.sum(e, axis=1, keepdims=True)
            outs.append(_mm_nt(vh, a))
        o = jnp.concatenate(outs, axis=0)
        t = t + _mm(w["tp"], o)
        t = t + _mm(w["tf2"], jnp.maximum(_mm(w["tf1"], t), 0.0))
        tokens = t

        kr = _mm(w["pk"], t)
        vr = _mm(w["pv"], t)
        kbs, vbs = [], []
        for b in range(B):
            kb, vb = _block_kv(kr[:, b * L:(b + 1) * L],
                               vr[:, b * L:(b + 1) * L], outc)
            kbs.append(kb)
            vbs.append(vb)
            k_out[b, r] = kb
            if outc < VPAD:
                vb_store = jnp.concatenate(
                    [vb, jnp.zeros((VPAD - outc, HEAD * L), jnp.float32)],
                    axis=0)
            else:
                vb_store = vb
            v_out[b, r] = vb_store
        kbig = jnp.concatenate(kbs, axis=1)
        vbig = jnp.concatenate(vbs, axis=1)

        fb = _mm(w["fb"], fs) if has_fb else fs
        qp = _mm(w["pq"], fb)
        logits = _mm_tt(kbig, qp)
        parts = []
        for g in range(HEAD * B):
            lg = logits[g * L:(g + 1) * L]
            lg = lg - jnp.max(lg, axis=0, keepdims=True)
            e = jnp.exp(lg)
            parts.append(e / jnp.sum(e, axis=0, keepdims=True))
        a = jnp.concatenate(parts, axis=0) * m_fs01
        fs = fb + _mm(vbig, a)


def _dense_kernel(x_ref, kall_ref, vall_ref, *refs):
    it = iter(refs[:-1])
    out_ref = refs[-1]
    stem1 = next(it)[...]
    stem2 = next(it)[...]
    pqs = [next(it)[...] for _ in RIM_SPECS]
    fbs = {r: next(it)[...] for r, s in enumerate(RIM_SPECS) if s[4]}
    cls1 = next(it)[...]
    cls2 = next(it)[...]

    xb = x_ref[0]
    f = _mm(stem2, jnp.maximum(_mm(stem1, xb), 0.0))
    for r, (_, _, inc, outc, has_fb, _) in enumerate(RIM_SPECS):
        kb = kall_ref[0, r]
        vb = vall_ref[0, r, 0:outc, :]
        if has_fb:
            q = _mm(_mm(pqs[r], fbs[r]), f)
            fb = _mm(fbs[r], f)
        else:
            fb = f
            q = _mm(pqs[r], fb)
        f = _point_attn(kb, vb, fb, q)
    out_ref[0] = _mm(cls2, jnp.maximum(_mm(cls1, f), 0.0))


def _flat_weights_a(params):
    ws = [params["stem1"], params["stem2"]]
    for (stage, vt, inc, outc, has_fb, _) in RIM_SPECS:
        p = params[stage][vt]
        ws.extend(p[kk] for kk in RIM_KEYS)
        if has_fb:
            ws.append(p["fb"])
    return ws


def _flat_weights_b(params):
    ws = [params["stem1"], params["stem2"]]
    ws.extend(params[s][vt]["pq"] for (s, vt, *_r) in RIM_SPECS)
    ws.extend(params[s][vt]["fb"]
              for (s, vt, _i, _o, has_fb, _t) in RIM_SPECS if has_fb)
    ws.extend([params["cls1"], params["cls2"]])
    return ws


def _full_spec(arr):
    nd = arr.ndim
    return pl.BlockSpec(arr.shape, lambda *_: (0,) * nd)


def kernel(x, params):
    coords_folded = x[:, :3, :].reshape(B, 3, FOLD, NF)
    wa = _flat_weights_a(params)
    in_specs_a = [
        pl.BlockSpec((B, CIN, N), lambda: (0, 0, 0)),
        pl.BlockSpec((B, 3, FOLD, NF), lambda: (0, 0, 0, 0)),
    ]
    in_specs_a += [_full_spec(w) for w in wa]
    nr = len(RIM_SPECS)
    k_all, v_all = pl.pallas_call(
        _tokens_kernel,
        grid=(),
        in_specs=in_specs_a,
        out_specs=[
            pl.BlockSpec((B, nr, TOKEN_C, HEAD * L), lambda: (0, 0, 0, 0)),
            pl.BlockSpec((B, nr, VPAD, HEAD * L), lambda: (0, 0, 0, 0)),
        ],
        out_shape=[
            jax.ShapeDtypeStruct((B, nr, TOKEN_C, HEAD * L), jnp.float32),
            jax.ShapeDtypeStruct((B, nr, VPAD, HEAD * L), jnp.float32),
        ],
    )(x, coords_folded, *wa)

    wb = _flat_weights_b(params)
    nt = N // TILE_N
    in_specs_b = [
        pl.BlockSpec((1, CIN, TILE_N), lambda b, t: (b, 0, t)),
        pl.BlockSpec((1, nr, TOKEN_C, HEAD * L), lambda b, t: (b, 0, 0, 0)),
        pl.BlockSpec((1, nr, VPAD, HEAD * L), lambda b, t: (b, 0, 0, 0)),
    ]
    in_specs_b += [_full_spec(w) for w in wb]
    out = pl.pallas_call(
        _dense_kernel,
        grid=(B, nt),
        in_specs=in_specs_b,
        out_specs=pl.BlockSpec((1, NUM_CLASSES, TILE_N), lambda b, t: (b, 0, t)),
        out_shape=jax.ShapeDtypeStruct((B, NUM_CLASSES, N), jnp.float32),
        compiler_params=pltpu.CompilerParams(
            dimension_semantics=("arbitrary", "arbitrary")),
    )(x, k_all, v_all, *wb)
    return out
